# baseline (speedup 1.0000x reference)
import functools

import jax
import jax.numpy as jnp
import numpy as np
from jax.experimental import pallas as pl

_FEAT_DIMS = [119, 10, 11, 12, 9, 5, 8, 2, 2]
_EDGE_DIMS = [22, 6, 2]


def _fused_body(x_ref, tab_ref, w1_ref, b1_ref, g1_ref, bb1_ref,
                w2_ref, b2_ref, g2_ref, bb2_ref, o_ref,
                *, offsets, vocab_pad, block_rows):
    idx = x_ref[...]
    iota = jax.lax.broadcasted_iota(jnp.int32, (block_rows, vocab_pad), 1)
    oh = jnp.zeros((block_rows, vocab_pad), jnp.bfloat16)
    for i, off in enumerate(offsets):
        oh = oh + (iota == idx[:, i][:, None] + off).astype(jnp.bfloat16)
    emb = jnp.dot(oh, tab_ref[...].astype(jnp.bfloat16),
                  preferred_element_type=jnp.float32)

    h = jnp.dot(emb.astype(jnp.bfloat16), w1_ref[...].astype(jnp.bfloat16),
                preferred_element_type=jnp.float32)
    h = h + b1_ref[...]
    mu = jnp.mean(h, axis=-1, keepdims=True)
    var = jnp.mean((h - mu) ** 2, axis=-1, keepdims=True)
    h = (h - mu) * jax.lax.rsqrt(var + 1e-5) * g1_ref[...] + bb1_ref[...]
    h = jax.nn.gelu(h)

    out = jnp.dot(h.astype(jnp.bfloat16), w2_ref[...].astype(jnp.bfloat16),
                  preferred_element_type=jnp.float32)
    out = out + b2_ref[...]
    mu = jnp.mean(out, axis=-1, keepdims=True)
    var = jnp.mean((out - mu) ** 2, axis=-1, keepdims=True)
    o_ref[...] = (out - mu) * jax.lax.rsqrt(var + 1e-5) * g2_ref[...] + bb2_ref[...]


def _embed_mix(idx, tables, mixer, dims, vocab_pad, block_rows):
    n_rows, n_feat = idx.shape
    d = tables[0].shape[1]
    tab = jnp.concatenate(tables, axis=0)
    tab = jnp.pad(tab, ((0, vocab_pad - tab.shape[0]), (0, 0)))
    offsets = tuple(int(v) for v in np.concatenate([[0], np.cumsum(dims[:-1])]))

    grid = (n_rows // block_rows,)
    row_spec = lambda shape: pl.BlockSpec(shape, lambda i: (i, 0))
    rep_spec = lambda shape: pl.BlockSpec(shape, lambda i: (0, 0))

    body = functools.partial(_fused_body, offsets=offsets,
                             vocab_pad=vocab_pad, block_rows=block_rows)
    return pl.pallas_call(
        body,
        grid=grid,
        in_specs=[
            row_spec((block_rows, n_feat)),
            rep_spec((vocab_pad, d)),
            rep_spec((d, 2 * d)),
            rep_spec((1, 2 * d)),
            rep_spec((1, 2 * d)),
            rep_spec((1, 2 * d)),
            rep_spec((2 * d, d)),
            rep_spec((1, d)),
            rep_spec((1, d)),
            rep_spec((1, d)),
        ],
        out_specs=row_spec((block_rows, d)),
        out_shape=jax.ShapeDtypeStruct((n_rows, d), jnp.float32),
    )(idx, tab,
      mixer['W1'], mixer['b1'][None, :], mixer['ln1_g'][None, :],
      mixer['ln1_b'][None, :],
      mixer['W2'], mixer['b2'][None, :], mixer['ln2_g'][None, :],
      mixer['ln2_b'][None, :])


def kernel(x, edge_attr, atom_tables, atom_mixer, edge_tables, edge_mixer):
    x_embedding = _embed_mix(x, atom_tables, atom_mixer, _FEAT_DIMS,
                             vocab_pad=256, block_rows=1000)
    edge_embedding = _embed_mix(edge_attr, edge_tables, edge_mixer, _EDGE_DIMS,
                                vocab_pad=32, block_rows=4000)
    return (x_embedding, edge_embedding)


# speedup vs baseline: 1.8891x; 1.8891x over previous
import functools

import jax
import jax.numpy as jnp
import numpy as np
from jax.experimental import pallas as pl

_FEAT_DIMS = [119, 10, 11, 12, 9, 5, 8, 2, 2]
_EDGE_DIMS = [22, 6, 2]


def _mixer_math(emb, w1_ref, b1_ref, g1_ref, bb1_ref,
                w2_ref, b2_ref, g2_ref, bb2_ref):
    h = jnp.dot(emb.astype(jnp.bfloat16), w1_ref[...].astype(jnp.bfloat16),
                preferred_element_type=jnp.float32)
    h = h + b1_ref[...]
    mu = jnp.mean(h, axis=-1, keepdims=True)
    var = jnp.mean((h - mu) ** 2, axis=-1, keepdims=True)
    h = (h - mu) * jax.lax.rsqrt(var + 1e-5) * g1_ref[...] + bb1_ref[...]
    h = jax.nn.gelu(h)
    out = jnp.dot(h.astype(jnp.bfloat16), w2_ref[...].astype(jnp.bfloat16),
                  preferred_element_type=jnp.float32)
    out = out + b2_ref[...]
    mu = jnp.mean(out, axis=-1, keepdims=True)
    var = jnp.mean((out - mu) ** 2, axis=-1, keepdims=True)
    return (out - mu) * jax.lax.rsqrt(var + 1e-5) * g2_ref[...] + bb2_ref[...]


def _onehot(cols, n, dtype):
    iota = jax.lax.broadcasted_iota(jnp.int32, (cols.shape[0], n), 1)
    return (iota == cols[:, None]).astype(dtype)


def _atom_body(x_ref, tab_ref, w1_ref, b1_ref, g1_ref, bb1_ref,
               w2_ref, b2_ref, g2_ref, bb2_ref, o_ref,
               *, offsets, vocab_pad, block_rows):
    idx = x_ref[...]
    iota = jax.lax.broadcasted_iota(jnp.int32, (block_rows, vocab_pad), 1)
    oh = jnp.zeros((block_rows, vocab_pad), jnp.bfloat16)
    for i, off in enumerate(offsets):
        oh = oh + (iota == idx[:, i][:, None] + off).astype(jnp.bfloat16)
    emb = jnp.dot(oh, tab_ref[...].astype(jnp.bfloat16),
                  preferred_element_type=jnp.float32)
    o_ref[...] = _mixer_math(emb, w1_ref, b1_ref, g1_ref, bb1_ref,
                             w2_ref, b2_ref, g2_ref, bb2_ref)


def _edge_combo_body(tabs_ref, w1_ref, b1_ref, g1_ref, bb1_ref,
                     w2_ref, b2_ref, g2_ref, bb2_ref, o_ref,
                     *, offsets, dims, n_pad):
    r = jax.lax.broadcasted_iota(jnp.int32, (n_pad, 1), 0)[:, 0]
    d1, d2 = dims[1], dims[2]
    feats = (r // (d1 * d2), (r // d2) % d1, r % d2)
    vocab_pad = tabs_ref.shape[0]
    oh = jnp.zeros((n_pad, vocab_pad), jnp.bfloat16)
    for f, off in zip(feats, offsets):
        oh = oh + _onehot(f + off, vocab_pad, jnp.bfloat16)
    emb = jnp.dot(oh, tabs_ref[...].astype(jnp.bfloat16),
                  preferred_element_type=jnp.float32)
    o_ref[...] = _mixer_math(emb, w1_ref, b1_ref, g1_ref, bb1_ref,
                             w2_ref, b2_ref, g2_ref, bb2_ref)


def _edge_lookup_body(e_ref, combo_ref, o_ref, *, dims, n_pad, block_rows):
    idx = e_ref[...]
    flat = (idx[:, 0] * (dims[1] * dims[2]) + idx[:, 1] * dims[2]
            + idx[:, 2])
    oh = _onehot(flat, n_pad, jnp.bfloat16)
    o_ref[...] = jnp.dot(oh, combo_ref[...].astype(jnp.bfloat16),
                         preferred_element_type=jnp.float32)


def _rep(shape):
    return pl.BlockSpec(shape, lambda i: (0,) * len(shape))


def _row(shape):
    return pl.BlockSpec(shape, lambda i: (i,) + (0,) * (len(shape) - 1))


def _mixer_args(mixer):
    return (mixer['W1'], mixer['b1'][None, :], mixer['ln1_g'][None, :],
            mixer['ln1_b'][None, :], mixer['W2'], mixer['b2'][None, :],
            mixer['ln2_g'][None, :], mixer['ln2_b'][None, :])


def _mixer_specs(d):
    return [_rep((d, 2 * d)), _rep((1, 2 * d)), _rep((1, 2 * d)),
            _rep((1, 2 * d)), _rep((2 * d, d)), _rep((1, d)),
            _rep((1, d)), _rep((1, d))]


def kernel(x, edge_attr, atom_tables, atom_mixer, edge_tables, edge_mixer):
    hn = atom_tables[0].shape[1]
    n_nodes, n_feat = x.shape
    atab = jnp.concatenate(atom_tables, axis=0)
    atab = jnp.pad(atab, ((0, 256 - atab.shape[0]), (0, 0)))
    a_off = tuple(int(v) for v in
                  np.concatenate([[0], np.cumsum(_FEAT_DIMS[:-1])]))
    bn = 1000
    x_embedding = pl.pallas_call(
        functools.partial(_atom_body, offsets=a_off, vocab_pad=256,
                          block_rows=bn),
        grid=(n_nodes // bn,),
        in_specs=[_row((bn, n_feat)), _rep((256, hn))] + _mixer_specs(hn),
        out_specs=_row((bn, hn)),
        out_shape=jax.ShapeDtypeStruct((n_nodes, hn), jnp.float32),
    )(x, atab, *_mixer_args(atom_mixer))

    he = edge_tables[0].shape[1]
    n_edges = edge_attr.shape[0]
    n_combo = int(np.prod(_EDGE_DIMS))
    n_pad = 384
    etab = jnp.concatenate(edge_tables, axis=0)
    etab = jnp.pad(etab, ((0, 32 - etab.shape[0]), (0, 0)))
    e_off = tuple(int(v) for v in
                  np.concatenate([[0], np.cumsum(_EDGE_DIMS[:-1])]))
    combo = pl.pallas_call(
        functools.partial(_edge_combo_body, offsets=e_off, dims=_EDGE_DIMS,
                          n_pad=n_pad),
        grid=(1,),
        in_specs=[_rep((32, he))] + _mixer_specs(he),
        out_specs=_rep((n_pad, he)),
        out_shape=jax.ShapeDtypeStruct((n_pad, he), jnp.float32),
    )(etab, *_mixer_args(edge_mixer))

    be = 4000
    edge_embedding = pl.pallas_call(
        functools.partial(_edge_lookup_body, dims=_EDGE_DIMS, n_pad=n_pad,
                          block_rows=be),
        grid=(n_edges // be,),
        in_specs=[_row((be, 3)), _rep((n_pad, he))],
        out_specs=_row((be, he)),
        out_shape=jax.ShapeDtypeStruct((n_edges, he), jnp.float32),
    )(edge_attr, combo)
    return (x_embedding, edge_embedding)


# speedup vs baseline: 2.0765x; 1.0992x over previous
import functools

import jax
import jax.numpy as jnp
import numpy as np
from jax.experimental import pallas as pl

_FEAT_DIMS = [119, 10, 11, 12, 9, 5, 8, 2, 2]
_EDGE_DIMS = [22, 6, 2]


def _mixer_math(emb, w1_ref, b1_ref, g1_ref, bb1_ref,
                w2_ref, b2_ref, g2_ref, bb2_ref):
    h = jnp.dot(emb.astype(jnp.bfloat16), w1_ref[...].astype(jnp.bfloat16),
                preferred_element_type=jnp.float32)
    h = h + b1_ref[...]
    mu = jnp.mean(h, axis=-1, keepdims=True)
    var = jnp.mean((h - mu) ** 2, axis=-1, keepdims=True)
    h = (h - mu) * jax.lax.rsqrt(var + 1e-5) * g1_ref[...] + bb1_ref[...]
    h = jax.nn.gelu(h)
    out = jnp.dot(h.astype(jnp.bfloat16), w2_ref[...].astype(jnp.bfloat16),
                  preferred_element_type=jnp.float32)
    out = out + b2_ref[...]
    mu = jnp.mean(out, axis=-1, keepdims=True)
    var = jnp.mean((out - mu) ** 2, axis=-1, keepdims=True)
    return (out - mu) * jax.lax.rsqrt(var + 1e-5) * g2_ref[...] + bb2_ref[...]


def _onehot(cols, n, dtype):
    iota = jax.lax.broadcasted_iota(jnp.int32, (cols.shape[0], n), 1)
    return (iota == cols[:, None]).astype(dtype)


def _atom_body(x_ref, m_ref, c_ref, tab_ref, w1_ref, b1_ref, g1_ref, bb1_ref,
               w2_ref, b2_ref, g2_ref, bb2_ref, o_ref):
    vals = jnp.dot(x_ref[...].astype(jnp.bfloat16), m_ref[...],
                   preferred_element_type=jnp.float32)
    oh = (vals == c_ref[...]).astype(jnp.bfloat16)
    emb = jnp.dot(oh, tab_ref[...].astype(jnp.bfloat16),
                  preferred_element_type=jnp.float32)
    o_ref[...] = _mixer_math(emb, w1_ref, b1_ref, g1_ref, bb1_ref,
                             w2_ref, b2_ref, g2_ref, bb2_ref)


def _edge_combo_body(tabs_ref, w1_ref, b1_ref, g1_ref, bb1_ref,
                     w2_ref, b2_ref, g2_ref, bb2_ref, o_ref,
                     *, offsets, dims, n_pad):
    r = jax.lax.broadcasted_iota(jnp.int32, (n_pad, 1), 0)[:, 0]
    d1, d2 = dims[1], dims[2]
    feats = (r // (d1 * d2), (r // d2) % d1, r % d2)
    vocab_pad = tabs_ref.shape[0]
    oh = jnp.zeros((n_pad, vocab_pad), jnp.bfloat16)
    for f, off in zip(feats, offsets):
        oh = oh + _onehot(f + off, vocab_pad, jnp.bfloat16)
    emb = jnp.dot(oh, tabs_ref[...].astype(jnp.bfloat16),
                  preferred_element_type=jnp.float32)
    o_ref[...] = _mixer_math(emb, w1_ref, b1_ref, g1_ref, bb1_ref,
                             w2_ref, b2_ref, g2_ref, bb2_ref)


def _edge_lookup_body(e_ref, m_ref, c_ref, combo_ref, o_ref):
    vals = jnp.dot(e_ref[...].astype(jnp.bfloat16), m_ref[...],
                   preferred_element_type=jnp.float32)
    oh = (vals == c_ref[...]).astype(jnp.bfloat16)
    o_ref[...] = jnp.dot(oh, combo_ref[...].astype(jnp.bfloat16),
                         preferred_element_type=jnp.float32)


def _rep(shape):
    return pl.BlockSpec(shape, lambda i: (0,) * len(shape))


def _row(shape):
    return pl.BlockSpec(shape, lambda i: (i,) + (0,) * (len(shape) - 1))


def _mixer_args(mixer):
    return (mixer['W1'], mixer['b1'][None, :], mixer['ln1_g'][None, :],
            mixer['ln1_b'][None, :], mixer['W2'], mixer['b2'][None, :],
            mixer['ln2_g'][None, :], mixer['ln2_b'][None, :])


def _mixer_specs(d):
    return [_rep((d, 2 * d)), _rep((1, 2 * d)), _rep((1, 2 * d)),
            _rep((1, 2 * d)), _rep((2 * d, d)), _rep((1, d)),
            _rep((1, d)), _rep((1, d))]


def kernel(x, edge_attr, atom_tables, atom_mixer, edge_tables, edge_mixer):
    hn = atom_tables[0].shape[1]
    n_nodes, n_feat = x.shape
    atab = jnp.concatenate(atom_tables, axis=0)
    atab = jnp.pad(atab, ((0, 256 - atab.shape[0]), (0, 0)))
    a_off = np.concatenate([[0], np.cumsum(_FEAT_DIMS[:-1])]).astype(np.int64)
    m_a = np.zeros((n_feat, 256), np.float32)
    c_a = np.full((1, 256), -1.0, np.float32)
    for i, (off, dim) in enumerate(zip(a_off, _FEAT_DIMS)):
        m_a[i, off:off + dim] = 1.0
        c_a[0, off:off + dim] = np.arange(dim, dtype=np.float32)
    bn = 1000
    x_embedding = pl.pallas_call(
        _atom_body,
        grid=(n_nodes // bn,),
        in_specs=[_row((bn, n_feat)), _rep((n_feat, 256)), _rep((1, 256)),
                  _rep((256, hn))] + _mixer_specs(hn),
        out_specs=_row((bn, hn)),
        out_shape=jax.ShapeDtypeStruct((n_nodes, hn), jnp.float32),
    )(x, jnp.asarray(m_a, jnp.bfloat16), jnp.asarray(c_a), atab,
      *_mixer_args(atom_mixer))

    he = edge_tables[0].shape[1]
    n_edges = edge_attr.shape[0]
    n_combo = int(np.prod(_EDGE_DIMS))
    n_pad = 384
    etab = jnp.concatenate(edge_tables, axis=0)
    etab = jnp.pad(etab, ((0, 32 - etab.shape[0]), (0, 0)))
    e_off = tuple(int(v) for v in
                  np.concatenate([[0], np.cumsum(_EDGE_DIMS[:-1])]))
    combo = pl.pallas_call(
        functools.partial(_edge_combo_body, offsets=e_off, dims=_EDGE_DIMS,
                          n_pad=n_pad),
        grid=(1,),
        in_specs=[_rep((32, he))] + _mixer_specs(he),
        out_specs=_rep((n_pad, he)),
        out_shape=jax.ShapeDtypeStruct((n_pad, he), jnp.float32),
    )(etab, *_mixer_args(edge_mixer))

    be = 4000
    m_e = np.tile(np.array([[_EDGE_DIMS[1] * _EDGE_DIMS[2]],
                            [_EDGE_DIMS[2]], [1]], np.float32), (1, n_pad))
    c_e = np.where(np.arange(n_pad) < n_combo,
                   np.arange(n_pad, dtype=np.float32), -1.0)[None, :]
    edge_embedding = pl.pallas_call(
        _edge_lookup_body,
        grid=(n_edges // be,),
        in_specs=[_row((be, 3)), _rep((3, n_pad)), _rep((1, n_pad)),
                  _rep((n_pad, he))],
        out_specs=_row((be, he)),
        out_shape=jax.ShapeDtypeStruct((n_edges, he), jnp.float32),
    )(edge_attr, jnp.asarray(m_e, jnp.bfloat16), jnp.asarray(c_e, jnp.float32),
      combo)
    return (x_embedding, edge_embedding)


# speedup vs baseline: 2.0835x; 1.0034x over previous
import functools

import jax
import jax.numpy as jnp
import numpy as np
from jax.experimental import pallas as pl
from jax.experimental.pallas import tpu as pltpu

_PARALLEL = pltpu.CompilerParams(dimension_semantics=("parallel",))

_FEAT_DIMS = [119, 10, 11, 12, 9, 5, 8, 2, 2]
_EDGE_DIMS = [22, 6, 2]


def _mixer_math(emb, w1_ref, b1_ref, g1_ref, bb1_ref,
                w2_ref, b2_ref, g2_ref, bb2_ref):
    h = jnp.dot(emb.astype(jnp.bfloat16), w1_ref[...].astype(jnp.bfloat16),
                preferred_element_type=jnp.float32)
    h = h + b1_ref[...]
    mu = jnp.mean(h, axis=-1, keepdims=True)
    var = jnp.mean((h - mu) ** 2, axis=-1, keepdims=True)
    h = (h - mu) * jax.lax.rsqrt(var + 1e-5) * g1_ref[...] + bb1_ref[...]
    h = jax.nn.gelu(h)
    out = jnp.dot(h.astype(jnp.bfloat16), w2_ref[...].astype(jnp.bfloat16),
                  preferred_element_type=jnp.float32)
    out = out + b2_ref[...]
    mu = jnp.mean(out, axis=-1, keepdims=True)
    var = jnp.mean((out - mu) ** 2, axis=-1, keepdims=True)
    return (out - mu) * jax.lax.rsqrt(var + 1e-5) * g2_ref[...] + bb2_ref[...]


def _onehot(cols, n, dtype):
    iota = jax.lax.broadcasted_iota(jnp.int32, (cols.shape[0], n), 1)
    return (iota == cols[:, None]).astype(dtype)


def _atom_body(x_ref, m_ref, c_ref, tab_ref, w1_ref, b1_ref, g1_ref, bb1_ref,
               w2_ref, b2_ref, g2_ref, bb2_ref, o_ref):
    vals = jnp.dot(x_ref[...].astype(jnp.bfloat16), m_ref[...],
                   preferred_element_type=jnp.float32)
    oh = (vals == c_ref[...]).astype(jnp.bfloat16)
    emb = jnp.dot(oh, tab_ref[...].astype(jnp.bfloat16),
                  preferred_element_type=jnp.float32)
    o_ref[...] = _mixer_math(emb, w1_ref, b1_ref, g1_ref, bb1_ref,
                             w2_ref, b2_ref, g2_ref, bb2_ref)


def _edge_combo_body(tabs_ref, w1_ref, b1_ref, g1_ref, bb1_ref,
                     w2_ref, b2_ref, g2_ref, bb2_ref, o_ref,
                     *, offsets, dims, n_pad):
    r = jax.lax.broadcasted_iota(jnp.int32, (n_pad, 1), 0)[:, 0]
    d1, d2 = dims[1], dims[2]
    feats = (r // (d1 * d2), (r // d2) % d1, r % d2)
    vocab_pad = tabs_ref.shape[0]
    oh = jnp.zeros((n_pad, vocab_pad), jnp.bfloat16)
    for f, off in zip(feats, offsets):
        oh = oh + _onehot(f + off, vocab_pad, jnp.bfloat16)
    emb = jnp.dot(oh, tabs_ref[...].astype(jnp.bfloat16),
                  preferred_element_type=jnp.float32)
    o_ref[...] = _mixer_math(emb, w1_ref, b1_ref, g1_ref, bb1_ref,
                             w2_ref, b2_ref, g2_ref, bb2_ref)


def _edge_lookup_body(e_ref, m_ref, c_ref, combo_ref, o_ref):
    vals = jnp.dot(e_ref[...].astype(jnp.bfloat16), m_ref[...],
                   preferred_element_type=jnp.float32)
    oh = (vals == c_ref[...]).astype(jnp.bfloat16)
    o_ref[...] = jnp.dot(oh, combo_ref[...].astype(jnp.bfloat16),
                         preferred_element_type=jnp.float32)


def _rep(shape):
    return pl.BlockSpec(shape, lambda i: (0,) * len(shape))


def _row(shape):
    return pl.BlockSpec(shape, lambda i: (i,) + (0,) * (len(shape) - 1))


def _mixer_args(mixer):
    return (mixer['W1'], mixer['b1'][None, :], mixer['ln1_g'][None, :],
            mixer['ln1_b'][None, :], mixer['W2'], mixer['b2'][None, :],
            mixer['ln2_g'][None, :], mixer['ln2_b'][None, :])


def _mixer_specs(d):
    return [_rep((d, 2 * d)), _rep((1, 2 * d)), _rep((1, 2 * d)),
            _rep((1, 2 * d)), _rep((2 * d, d)), _rep((1, d)),
            _rep((1, d)), _rep((1, d))]


def kernel(x, edge_attr, atom_tables, atom_mixer, edge_tables, edge_mixer):
    hn = atom_tables[0].shape[1]
    n_nodes, n_feat = x.shape
    atab = jnp.concatenate(atom_tables, axis=0)
    atab = jnp.pad(atab, ((0, 256 - atab.shape[0]), (0, 0)))
    a_off = np.concatenate([[0], np.cumsum(_FEAT_DIMS[:-1])]).astype(np.int64)
    m_a = np.zeros((n_feat, 256), np.float32)
    c_a = np.full((1, 256), -1.0, np.float32)
    for i, (off, dim) in enumerate(zip(a_off, _FEAT_DIMS)):
        m_a[i, off:off + dim] = 1.0
        c_a[0, off:off + dim] = np.arange(dim, dtype=np.float32)
    bn = 1000
    x_embedding = pl.pallas_call(
        _atom_body,
        grid=(n_nodes // bn,),
        in_specs=[_row((bn, n_feat)), _rep((n_feat, 256)), _rep((1, 256)),
                  _rep((256, hn))] + _mixer_specs(hn),
        out_specs=_row((bn, hn)),
        out_shape=jax.ShapeDtypeStruct((n_nodes, hn), jnp.float32),
        compiler_params=_PARALLEL,
    )(x, jnp.asarray(m_a, jnp.bfloat16), jnp.asarray(c_a), atab,
      *_mixer_args(atom_mixer))

    he = edge_tables[0].shape[1]
    n_edges = edge_attr.shape[0]
    n_combo = int(np.prod(_EDGE_DIMS))
    n_pad = 384
    etab = jnp.concatenate(edge_tables, axis=0)
    etab = jnp.pad(etab, ((0, 32 - etab.shape[0]), (0, 0)))
    e_off = tuple(int(v) for v in
                  np.concatenate([[0], np.cumsum(_EDGE_DIMS[:-1])]))
    combo = pl.pallas_call(
        functools.partial(_edge_combo_body, offsets=e_off, dims=_EDGE_DIMS,
                          n_pad=n_pad),
        grid=(1,),
        in_specs=[_rep((32, he))] + _mixer_specs(he),
        out_specs=_rep((n_pad, he)),
        out_shape=jax.ShapeDtypeStruct((n_pad, he), jnp.float32),
    )(etab, *_mixer_args(edge_mixer))

    be = 4000
    m_e = np.tile(np.array([[_EDGE_DIMS[1] * _EDGE_DIMS[2]],
                            [_EDGE_DIMS[2]], [1]], np.float32), (1, n_pad))
    c_e = np.where(np.arange(n_pad) < n_combo,
                   np.arange(n_pad, dtype=np.float32), -1.0)[None, :]
    edge_embedding = pl.pallas_call(
        _edge_lookup_body,
        grid=(n_edges // be,),
        in_specs=[_row((be, 3)), _rep((3, n_pad)), _rep((1, n_pad)),
                  _rep((n_pad, he))],
        out_specs=_row((be, he)),
        out_shape=jax.ShapeDtypeStruct((n_edges, he), jnp.float32),
        compiler_params=_PARALLEL,
    )(edge_attr, jnp.asarray(m_e, jnp.bfloat16), jnp.asarray(c_e, jnp.float32),
      combo)
    return (x_embedding, edge_embedding)
